# baseline (device time: 62507 ns/iter reference)
import jax
import jax.numpy as jnp
from jax import lax
from jax.experimental import pallas as pl
from jax.experimental.pallas import tpu as pltpu

B = 16
H = 16
D = 64
BS = 16
NP_LOC = 128
NB = 128
NKEY = NP_LOC * BS
SLOTS = 8
SCALE = D ** -0.5


def kernel(Q, K, V, bt, lens):
    lens2d = lens.reshape(B, 1)
    q_t = jnp.transpose(Q[:, 0, :, :], (1, 2, 0))

    def body(q_ref, k_hbm, v_hbm, bt_ref, lens_ref, out_ref,
             kbuf, vbuf, ckey_ref, comm_ref, ksems, vsems,
             send_sem, recv_sem):
        my_x = lax.axis_index("x")
        my_y = lax.axis_index("y")
        my_z = lax.axis_index("z")
        peer = (1 - my_x, my_y, my_z)

        kcps, vcps = [], []
        for h in range(H):
            slot = h % SLOTS
            kcps.append(pltpu.make_async_copy(
                k_hbm.at[:, :, h, :], kbuf.at[slot], ksems.at[slot]))
            vcps.append(pltpu.make_async_copy(
                v_hbm.at[:, :, h, :], vbuf.at[slot], vsems.at[slot]))
        for h in range(SLOTS):
            kcps[h].start()
            vcps[h].start()

        bt3 = bt_ref[:, :][:, :, None]
        pids = lax.broadcasted_iota(jnp.int32, (B, NB, NP_LOC), 2) \
            + my_x * NP_LOC
        jidx = lax.broadcasted_iota(jnp.int32, (B, NB, 1), 1)
        valid = jidx < lens_ref[:, :][:, None, :]
        hit = jnp.logical_and(bt3 == pids, valid)
        cnt = jnp.sum(jnp.where(hit, 1.0, 0.0), axis=1)
        ek = jnp.where(
            lax.broadcasted_iota(jnp.int32, (NP_LOC, NKEY), 0)
            == lax.broadcasted_iota(jnp.int32, (NP_LOC, NKEY), 1) // BS,
            1.0, 0.0).astype(jnp.bfloat16)
        ckey_ref[:, :] = lax.dot_general(
            cnt.astype(jnp.bfloat16), ek,
            (((1,), (0,)), ((), ())),
            preferred_element_type=jnp.float32)
        ck = ckey_ref[:, :]

        for h in range(H):
            kcps[h].wait()
            vcps[h].wait()
            slot = h % SLOTS
            k_h = kbuf[slot].reshape(NKEY, D).astype(jnp.bfloat16)
            v_h = vbuf[slot].reshape(NKEY, D).astype(jnp.bfloat16)
            q_ht = q_ref[h].astype(jnp.bfloat16)
            s_t = lax.dot_general(
                k_h, q_ht, (((1,), (0,)), ((), ())),
                preferred_element_type=jnp.float32,
            ) * SCALE
            s = jnp.transpose(s_t)
            s = jnp.where(ck > 0.0, s, -1e30)
            m = jnp.max(s, axis=1, keepdims=True)
            p = ck * jnp.exp(s - m)
            l = jnp.sum(p, axis=1, keepdims=True)
            acc = lax.dot_general(
                p.astype(jnp.bfloat16), v_h, (((1,), (0,)), ((), ())),
                preferred_element_type=jnp.float32,
            )
            comm_ref[0, h, :, 0:D] = acc
            comm_ref[0, h, :, D:D + 1] = m
            comm_ref[0, h, :, D + 1:D + 2] = l
            if h + SLOTS < H:
                kcps[h + SLOTS].start()
                vcps[h + SLOTS].start()

        barrier_sem = pltpu.get_barrier_semaphore()
        pl.semaphore_signal(barrier_sem, inc=1, device_id=peer,
                            device_id_type=pl.DeviceIdType.MESH)
        pl.semaphore_wait(barrier_sem, 1)

        rdma = pltpu.make_async_remote_copy(
            src_ref=comm_ref.at[0],
            dst_ref=comm_ref.at[1],
            send_sem=send_sem,
            recv_sem=recv_sem,
            device_id=peer,
            device_id_type=pl.DeviceIdType.MESH,
        )
        rdma.start()
        rdma.wait()

        acc1 = comm_ref[0, :, :, 0:D]
        m1 = comm_ref[0, :, :, D:D + 1]
        l1 = comm_ref[0, :, :, D + 1:D + 2]
        acc2 = comm_ref[1, :, :, 0:D]
        m2 = comm_ref[1, :, :, D:D + 1]
        l2 = comm_ref[1, :, :, D + 1:D + 2]
        m_new = jnp.maximum(m1, m2)
        a1 = jnp.exp(m1 - m_new)
        a2 = jnp.exp(m2 - m_new)
        l_tot = l1 * a1 + l2 * a2
        res = (acc1 * a1 + acc2 * a2) / l_tot
        out_ref[:, 0, :, :] = jnp.transpose(res, (1, 0, 2))

    return pl.pallas_call(
        body,
        out_shape=jax.ShapeDtypeStruct((B, 1, H, D), jnp.float32),
        in_specs=[
            pl.BlockSpec(memory_space=pltpu.VMEM),
            pl.BlockSpec(memory_space=pl.ANY),
            pl.BlockSpec(memory_space=pl.ANY),
            pl.BlockSpec(memory_space=pltpu.VMEM),
            pl.BlockSpec(memory_space=pltpu.VMEM),
        ],
        out_specs=pl.BlockSpec(memory_space=pltpu.VMEM),
        scratch_shapes=[
            pltpu.VMEM((SLOTS, NP_LOC, BS, D), jnp.float32),
            pltpu.VMEM((SLOTS, NP_LOC, BS, D), jnp.float32),
            pltpu.VMEM((B, NKEY), jnp.float32),
            pltpu.VMEM((2, H, B, 128), jnp.float32),
            pltpu.SemaphoreType.DMA((SLOTS,)),
            pltpu.SemaphoreType.DMA((SLOTS,)),
            pltpu.SemaphoreType.DMA,
            pltpu.SemaphoreType.DMA,
        ],
        compiler_params=pltpu.CompilerParams(collective_id=0),
    )(q_t, K, V, bt, lens2d)


# device time: 39367 ns/iter; 1.5878x vs baseline; 1.5878x over previous
import jax
import jax.numpy as jnp
from jax import lax
from jax.experimental import pallas as pl
from jax.experimental.pallas import tpu as pltpu

B = 16
H = 16
D = 64
BS = 16
NP_LOC = 128
NB = 128
NKEY = NP_LOC * BS
SCALE = D ** -0.5


def kernel(Q, K, V, bt, lens):
    lens2d = lens.reshape(B, 1)
    q_t = jnp.transpose(Q[:, 0, :, :], (1, 0, 2)).astype(jnp.bfloat16)
    k_dt = jnp.transpose(K.reshape(NKEY, H, D), (1, 2, 0)).astype(jnp.bfloat16)
    v_t = jnp.transpose(V.reshape(NKEY, H, D), (1, 0, 2)).astype(jnp.bfloat16)

    def body(q_ref, k_ref, v_ref, bt_ref, lens_ref, out_ref,
             comm_ref, send_sem, recv_sem):
        my_x = lax.axis_index("x")
        my_y = lax.axis_index("y")
        my_z = lax.axis_index("z")
        peer = (1 - my_x, my_y, my_z)

        bt3 = bt_ref[:, :][:, :, None]
        pids = lax.broadcasted_iota(jnp.int32, (B, NB, NP_LOC), 2) \
            + my_x * NP_LOC
        jidx = lax.broadcasted_iota(jnp.int32, (B, NB, 1), 1)
        valid = jidx < lens_ref[:, :][:, None, :]
        hit = jnp.logical_and(bt3 == pids, valid)
        cnt = jnp.sum(jnp.where(hit, 1.0, 0.0), axis=1)
        ek = jnp.where(
            lax.broadcasted_iota(jnp.int32, (NP_LOC, NKEY), 0)
            == lax.broadcasted_iota(jnp.int32, (NP_LOC, NKEY), 1) // BS,
            1.0, 0.0).astype(jnp.bfloat16)
        ck = lax.dot_general(
            cnt.astype(jnp.bfloat16), ek,
            (((1,), (0,)), ((), ())),
            preferred_element_type=jnp.float32)

        s = lax.dot_general(
            q_ref[:, :, :], k_ref[:, :, :],
            (((2,), (1,)), ((0,), (0,))),
            preferred_element_type=jnp.float32,
        ) * SCALE
        s = jnp.where(ck[None, :, :] > 0.0, s, -1e30)
        m = jnp.max(s, axis=2, keepdims=True)
        p = ck[None, :, :] * jnp.exp(s - m)
        l = jnp.sum(p, axis=2, keepdims=True)
        acc = lax.dot_general(
            p.astype(jnp.bfloat16), v_ref[:, :, :],
            (((2,), (1,)), ((0,), (0,))),
            preferred_element_type=jnp.float32,
        )

        comm_ref[0, :, :, 0:D] = acc
        comm_ref[0, :, :, D:D + 1] = m
        comm_ref[0, :, :, D + 1:D + 2] = l

        barrier_sem = pltpu.get_barrier_semaphore()
        pl.semaphore_signal(barrier_sem, inc=1, device_id=peer,
                            device_id_type=pl.DeviceIdType.MESH)
        pl.semaphore_wait(barrier_sem, 1)

        rdma = pltpu.make_async_remote_copy(
            src_ref=comm_ref.at[0],
            dst_ref=comm_ref.at[1],
            send_sem=send_sem,
            recv_sem=recv_sem,
            device_id=peer,
            device_id_type=pl.DeviceIdType.MESH,
        )
        rdma.start()
        rdma.wait()

        acc1 = comm_ref[0, :, :, 0:D]
        m1 = comm_ref[0, :, :, D:D + 1]
        l1 = comm_ref[0, :, :, D + 1:D + 2]
        acc2 = comm_ref[1, :, :, 0:D]
        m2 = comm_ref[1, :, :, D:D + 1]
        l2 = comm_ref[1, :, :, D + 1:D + 2]
        m_new = jnp.maximum(m1, m2)
        a1 = jnp.exp(m1 - m_new)
        a2 = jnp.exp(m2 - m_new)
        l_tot = l1 * a1 + l2 * a2
        res = (acc1 * a1 + acc2 * a2) / l_tot
        out_ref[:, 0, :, :] = jnp.transpose(res, (1, 0, 2))

    return pl.pallas_call(
        body,
        out_shape=jax.ShapeDtypeStruct((B, 1, H, D), jnp.float32),
        in_specs=[
            pl.BlockSpec(memory_space=pltpu.VMEM),
            pl.BlockSpec(memory_space=pltpu.VMEM),
            pl.BlockSpec(memory_space=pltpu.VMEM),
            pl.BlockSpec(memory_space=pltpu.VMEM),
            pl.BlockSpec(memory_space=pltpu.VMEM),
        ],
        out_specs=pl.BlockSpec(memory_space=pltpu.VMEM),
        scratch_shapes=[
            pltpu.VMEM((2, H, B, 128), jnp.float32),
            pltpu.SemaphoreType.DMA,
            pltpu.SemaphoreType.DMA,
        ],
        compiler_params=pltpu.CompilerParams(collective_id=0),
    )(q_t, k_dt, v_t, bt, lens2d)
